# Initial kernel scaffold; baseline (speedup 1.0000x reference)
import jax
import jax.numpy as jnp
from jax.experimental import pallas as pl


def kernel(input_labels, pos_labels, neg_labels, W_in, W_out):
    raise NotImplementedError("write your pallas kernel here")


# speedup vs baseline: 7.6828x; 7.6828x over previous
import functools

import jax
import jax.numpy as jnp
from jax import lax
from jax.experimental import pallas as pl
from jax.experimental.pallas import tpu as pltpu
from jax.experimental.pallas import tpu_sc as plsc

WALK = 10
E = 128
B = 16384
NC = 2
NS = 16
NW = NC * NS
C = 128
K = 4

IN_CH = B // (NW * C)
POS_CH = B * WALK // (NW * C)
NEG_CH = B * WALK * 5 // (NW * C)


def _emb_body(in_idx, pos_idx, neg_idx, w_in, w_out, o_in, o_pos, o_neg,
              in_v, pos_v, neg_v, b0, b1, b2, b3,
              g0, g1, g2, g3, s0, s1, s2, s3):
    bufs = (b0, b1, b2, b3)
    gsems = (g0, g1, g2, g3)
    ssems = (s0, s1, s2, s3)
    wid = lax.axis_index("s") * NC + lax.axis_index("c")

    pltpu.sync_copy(in_idx.at[wid], in_v)
    pltpu.sync_copy(pos_idx.at[wid], pos_v)
    pltpu.sync_copy(neg_idx.at[wid], neg_v)

    in_base = wid * (B // NW)
    ghs = [pltpu.async_copy(w_in.at[in_v.at[j]], bufs[j], gsems[j])
           for j in range(IN_CH)]
    shs = []
    for j in range(IN_CH):
        ghs[j].wait()
        for k in range(WALK):
            shs.append(pltpu.async_copy(
                bufs[j], o_in.at[pl.ds(k * B + in_base + j * C, C)], ssems[j]))
    for h in shs:
        h.wait()

    def run_phase(idx_v, out, nch, base_row):
        def group(i, carry):
            gh = [pltpu.async_copy(w_out.at[idx_v.at[i * K + b]], bufs[b],
                                   gsems[b])
                  for b in range(K)]
            sh = []
            for b in range(K):
                gh[b].wait()
                row0 = base_row + (i * K + b) * C
                sh.append(pltpu.async_copy(bufs[b], out.at[pl.ds(row0, C)],
                                           ssems[b]))
            for h in sh:
                h.wait()
            return carry
        lax.fori_loop(0, nch // K, group, 0)

    run_phase(pos_v, o_pos, POS_CH, wid * POS_CH * C)
    run_phase(neg_v, o_neg, NEG_CH, wid * NEG_CH * C)


_emb = functools.partial(
    pl.kernel,
    mesh=plsc.VectorSubcoreMesh(core_axis_name="c", subcore_axis_name="s"),
    out_type=(
        jax.ShapeDtypeStruct((B * WALK, E), jnp.float32),
        jax.ShapeDtypeStruct((B * WALK, E), jnp.float32),
        jax.ShapeDtypeStruct((B * WALK * 5, E), jnp.float32),
    ),
    scratch_types=[
        pltpu.VMEM((IN_CH, C), jnp.int32),
        pltpu.VMEM((POS_CH, C), jnp.int32),
        pltpu.VMEM((NEG_CH, C), jnp.int32),
    ] + [pltpu.VMEM((C, E), jnp.float32) for _ in range(K)]
      + [pltpu.SemaphoreType.DMA for _ in range(2 * K)],
)(_emb_body)


def kernel(input_labels, pos_labels, neg_labels, W_in, W_out):
    in_idx = input_labels.reshape(NW, IN_CH, C).astype(jnp.int32)
    pos_idx = pos_labels.reshape(NW, POS_CH, C).astype(jnp.int32)
    neg_idx = neg_labels.reshape(NW, NEG_CH, C).astype(jnp.int32)
    return _emb(in_idx, pos_idx, neg_idx, W_in, W_out)
